# baseline (speedup 1.0000x reference)
import functools

import jax
import jax.numpy as jnp
from jax import lax
from jax.experimental import pallas as pl
from jax.experimental.pallas import tpu as pltpu
from jax.experimental.pallas import tpu_sc as plsc

_L = 16


def _make_sc_kernel(V, D, B):
    NW = 32
    bpw = B // NW
    n_rows = 2 * bpw
    HR = n_rows // 2
    IDXW = 128
    n_dma = HR // IDXW
    n_grp = HR // 2 // _L
    DC = D // _L
    half = V // 2

    mesh = plsc.VectorSubcoreMesh(core_axis_name="c", subcore_axis_name="s")

    @functools.partial(
        pl.kernel,
        out_type=jax.ShapeDtypeStruct((B,), jnp.float32),
        mesh=mesh,
        scratch_types=[
            pltpu.VMEM((n_dma, IDXW), jnp.int32),
            pltpu.VMEM((n_dma, IDXW), jnp.int32),
            pltpu.VMEM((HR,), jnp.int32),
            pltpu.VMEM((HR, D), jnp.float32),
            pltpu.VMEM((HR, D), jnp.float32),
            pltpu.VMEM((D,), jnp.float32),
            pltpu.VMEM((_L, _L), jnp.float32),
            pltpu.VMEM((bpw,), jnp.float32),
            pltpu.SemaphoreType.DMA,
        ],
        compiler_params=pltpu.CompilerParams(
            needs_layout_passes=False, use_tc_tiling_on_sc=False
        ),
    )
    def run(top_hbm, bot_hbm, idx_hbm, r_hbm, out_hbm,
            idxt_v, idxb_v, idx_v, rows_t, rows_b, r_v, p_v, out_v, sem):
        wid = lax.axis_index("s") * 2 + lax.axis_index("c")
        pltpu.sync_copy(r_hbm, r_v)
        r_regs = [r_v[pl.ds(c * _L, _L)] for c in range(DC)]
        iota = lax.iota(jnp.int32, _L)

        for hs in range(2):
            base = wid * n_rows + hs * HR
            pltpu.sync_copy(idx_hbm.at[pl.ds(base, HR)], idx_v)

            for t in range(HR // _L):
                vv = idx_v[pl.ds(t * _L, _L)]
                m = vv < half
                zt = jnp.where(m, vv, 0)
                zb = jnp.where(m, 0, vv - half)
                r2 = t // (IDXW // _L)
                c2 = (t % (IDXW // _L)) * _L
                idxt_v[r2, pl.ds(c2, _L)] = zt
                idxb_v[r2, pl.ds(c2, _L)] = zb

            copies = []
            for j in range(n_dma):
                copies.append(pltpu.make_async_copy(
                    top_hbm.at[idxt_v.at[j]],
                    rows_t.at[pl.ds(j * IDXW, IDXW)],
                    sem,
                ))
                copies.append(pltpu.make_async_copy(
                    bot_hbm.at[idxb_v.at[j]],
                    rows_b.at[pl.ds(j * IDXW, IDXW)],
                    sem,
                ))
            for c in copies:
                c.start()
            for c in copies:
                c.wait()

            def group_body(g, carry):
                vv0 = idx_v[pl.ds(2 * g * _L, _L)]
                vv1 = idx_v[pl.ds((2 * g + 1) * _L, _L)]
                for j in range(_L):
                    i2 = 2 * (g * _L + j)
                    vs = (vv0, vv1)[(2 * j) // _L][(2 * j) % _L]
                    vo = (vv0, vv1)[(2 * j + 1) // _L][(2 * j + 1) % _L]
                    ms = jnp.broadcast_to(vs < half, (_L,))
                    mo = jnp.broadcast_to(vo < half, (_L,))
                    acc = None
                    for c in range(DC):
                        st = rows_t[i2, pl.ds(c * _L, _L)]
                        sb = rows_b[i2, pl.ds(c * _L, _L)]
                        ot = rows_t[i2 + 1, pl.ds(c * _L, _L)]
                        ob = rows_b[i2 + 1, pl.ds(c * _L, _L)]
                        s_c = jnp.where(ms, st, sb)
                        o_c = jnp.where(mo, ot, ob)
                        t2 = (s_c * o_c) * r_regs[c]
                        acc = t2 if acc is None else acc + t2
                    p_v[j, :] = acc
                accv = jnp.zeros((_L,), jnp.float32)
                for l in range(_L):
                    col = plsc.load_gather(p_v, [iota, jnp.full((_L,), l, jnp.int32)])
                    accv = accv + col
                sig = 1.0 / (1.0 + jnp.exp(-accv))
                out_v[pl.ds(hs * HR // 2 + g * _L, _L)] = sig
                return carry

            lax.fori_loop(0, n_grp, group_body, 0, unroll=False)

        pltpu.sync_copy(out_v, out_hbm.at[pl.ds(wid * bpw, bpw)])

    return run


def kernel(emb, batch_ind, r):
    V, D = emb.shape
    B = batch_ind.shape[0]
    half = V // 2
    top = emb[:half]
    bot = emb[half:]
    idx_flat = batch_ind.reshape(2 * B)
    run = _make_sc_kernel(V, D, B)
    return run(top, bot, idx_flat, r)


# speedup vs baseline: 2.1503x; 2.1503x over previous
import functools

import jax
import jax.numpy as jnp
from jax import lax
from jax.experimental import pallas as pl
from jax.experimental.pallas import tpu as pltpu
from jax.experimental.pallas import tpu_sc as plsc

_L = 16


def _make_sc_kernel(V, D, B):
    NW = 32
    bpw = B // NW
    n_rows = 2 * bpw
    IDXW = 128
    HR = n_rows // 2
    n_dma = HR // IDXW
    n_grp = bpw // _L
    DC = D // _L

    mesh = plsc.VectorSubcoreMesh(core_axis_name="c", subcore_axis_name="s")

    @functools.partial(
        pl.kernel,
        out_type=jax.ShapeDtypeStruct((B,), jnp.float32),
        mesh=mesh,
        scratch_types=[
            pltpu.VMEM((n_dma, IDXW), jnp.int32),
            pltpu.VMEM((HR, 2 * D), jnp.float32),
            pltpu.VMEM((D,), jnp.float32),
            pltpu.VMEM((_L, _L), jnp.float32),
            pltpu.VMEM((bpw,), jnp.float32),
            pltpu.SemaphoreType.DMA,
        ],
        compiler_params=pltpu.CompilerParams(needs_layout_passes=False),
    )
    def run(emb_hbm, idx_hbm, r_hbm, out_hbm, idx_v, rows_v, r_v, p_v, out_v, sem):
        wid = lax.axis_index("s") * 2 + lax.axis_index("c")
        pltpu.sync_copy(r_hbm, r_v)
        r_regs = [r_v[pl.ds(c * _L, _L)] for c in range(DC)]
        iota = lax.iota(jnp.int32, _L)

        for hs in range(2):
            base = wid * n_rows + hs * HR
            for j in range(n_dma):
                pltpu.sync_copy(idx_hbm.at[pl.ds(base + j * IDXW, IDXW)], idx_v.at[j])

            copies = [
                pltpu.make_async_copy(
                    emb_hbm.at[idx_v.at[j]],
                    rows_v.at[pl.ds(j * IDXW, IDXW)],
                    sem,
                )
                for j in range(n_dma)
            ]
            for c in copies:
                c.start()
            for c in copies:
                c.wait()

            def group_body(g, carry):
                row0 = g * _L
                for j in range(_L):
                    i2 = 2 * (row0 + j)
                    acc = None
                    for c in range(DC):
                        s_c = rows_v[i2, pl.ds(c * _L, _L)]
                        o_c = rows_v[i2 + 1, pl.ds(c * _L, _L)]
                        t = (s_c * o_c) * r_regs[c]
                        acc = t if acc is None else acc + t
                    p_v[j, :] = acc
                accv = jnp.zeros((_L,), jnp.float32)
                for l in range(_L):
                    col = plsc.load_gather(p_v, [iota, jnp.full((_L,), l, jnp.int32)])
                    accv = accv + col
                sig = 1.0 / (1.0 + jnp.exp(-accv))
                out_v[pl.ds(hs * (bpw // 2) + g * _L, _L)] = sig
                return carry

            lax.fori_loop(0, n_grp // 2, group_body, 0, unroll=False)

        pltpu.sync_copy(out_v, out_hbm.at[pl.ds(wid * bpw, bpw)])

    return run


def kernel(emb, batch_ind, r):
    V, D = emb.shape
    B = batch_ind.shape[0]
    emb128 = jnp.pad(emb, ((0, 0), (0, D)))
    idx_flat = batch_ind.reshape(2 * B)
    run = _make_sc_kernel(V, D, B)
    return run(emb128, idx_flat, r)
